# baseline (speedup 1.0000x reference)
import functools

import jax
import jax.numpy as jnp
from jax import lax
from jax.experimental import pallas as pl
from jax.experimental.pallas import tpu as pltpu
from jax.experimental.pallas import tpu_sc as plsc

_B, _T, _S = 4096, 200, 128
_NW = 32
_CPW = _B // _NW
_NC = 2
_BAND = 8


def _make_sc_gather():
    mesh = plsc.VectorSubcoreMesh(core_axis_name="c", subcore_axis_name="s")

    @functools.partial(
        pl.kernel,
        out_type=jax.ShapeDtypeStruct((_T, _B), jnp.float32),
        mesh=mesh,
        compiler_params=pltpu.CompilerParams(use_tc_tiling_on_sc=True),
        scratch_types=[
            pltpu.VMEM((_T, _CPW), jnp.int32),
            pltpu.VMEM((_T, _CPW), jnp.int32),
            pltpu.VMEM((_T, _CPW), jnp.float32),
            pltpu.SemaphoreType.DMA,
        ],
    )
    def sc_gather(table_hbm, qst_hbm, out_hbm, qs_v, idx_v, val_v, sem):
        wid = lax.axis_index("s") * _NC + lax.axis_index("c")
        col0 = wid * _CPW

        pltpu.sync_copy(qst_hbm.at[:, pl.ds(col0, _CPW)], qs_v)

        lane_base = lax.iota(jnp.int32, 16) * (_T * _S)

        def compute_and_fire(band, carry):
            t0 = band * _BAND
            for dt in range(_BAND):
                t = t0 + dt
                for h in range(8):
                    s_ids = qs_v[t, pl.ds(h * 16, 16)] & (_S - 1)
                    base = (col0 + h * 16) * (_T * _S) + t * _S
                    idx_v[t, pl.ds(h * 16, 16)] = s_ids + (lane_base + base)
            pltpu.async_copy(
                table_hbm.at[
                    idx_v.reshape(_T // _BAND, _BAND * _CPW).at[band]],
                val_v.reshape(_T // _BAND, _BAND * _CPW).at[band],
                sem)
            return carry

        lax.fori_loop(0, _T // _BAND, compute_and_fire, 0)

        pltpu.make_async_copy(out_hbm.at[:, pl.ds(col0, _CPW)], val_v,
                              sem).wait()

        pltpu.sync_copy(val_v, out_hbm.at[:, pl.ds(col0, _CPW)])

    return sc_gather


_sc_gather = _make_sc_gather()


def kernel(mastery_levels, question_skills):
    table = jnp.reshape(mastery_levels, (-1,))
    qs_t = jnp.transpose(question_skills.astype(jnp.int32))
    out_t = _sc_gather(table, qs_t)
    return jnp.transpose(out_t)
